# baseline (speedup 1.0000x reference)
import functools

import jax
import jax.numpy as jnp
from jax import lax
from jax.experimental import pallas as pl
from jax.experimental.pallas import tpu as pltpu
from jax.experimental.pallas import tpu_sc as plsc

N = 50000
D = 128
S = 10
EMB = 128
B = 8192

_INFO = plsc.get_sparse_core_info()
_NC = _INFO.num_cores
_NS = _INFO.num_subcores
_NW = _NC * _NS
_B_PER_W = B // _NW
_CHUNK = 128
_HB = B // 2


def _sc_self_kernel(feat_hbm, nodes_hbm, self_out,
                    nodes_v, self_v, sem_g, sem_w):
  wid = lax.axis_index("s") * _NC + lax.axis_index("c")
  pltpu.sync_copy(nodes_hbm.at[pl.ds(wid * 2, 2)], nodes_v)
  cps = [pltpu.async_copy(feat_hbm.at[nodes_v.at[c]], self_v.at[c], sem_g)
         for c in range(2)]
  outs = []
  for c in range(2):
    cps[c].wait()
    outs.append(pltpu.async_copy(
        self_v.at[c], self_out.at[pl.ds(wid * _B_PER_W + c * _CHUNK, _CHUNK)],
        sem_w))
  for cp in outs:
    cp.wait()


def _make_nsum_kernel(half):
  def body(feat_hbm, nbf_hbm, nsum_out,
           nbf_v, nbt_v, nsum_v, sem_nb, sem_s0, sem_acc, sem_out):
    wid = lax.axis_index("s") * _NC + lax.axis_index("c")
    gbase = half * _HB + wid * _CHUNK
    pltpu.sync_copy(nbf_hbm.at[pl.ds(gbase * S, _CHUNK * S)], nbf_v)
    for s in range(S):
      for g in range(_CHUNK // 16):
        idx = lax.iota(jnp.int32, 16) * S + jnp.int32(g * 16 * S + s)
        nbt_v[s, pl.ds(g * 16, 16)] = plsc.load_gather(nbf_v, [idx])
    pltpu.async_copy(feat_hbm.at[nbt_v.at[0]], nsum_v, sem_s0).wait()
    cps = [pltpu.async_copy(feat_hbm.at[nbt_v.at[s]], nsum_v, sem_acc,
                            add=True) for s in range(1, S)]
    for cp in cps:
      cp.wait()
    pltpu.async_copy(nsum_v, nsum_out.at[pl.ds(wid * _CHUNK, _CHUNK)],
                     sem_out).wait()

  mesh = plsc.VectorSubcoreMesh(core_axis_name="c", subcore_axis_name="s")
  return pl.kernel(
      body,
      out_type=jax.ShapeDtypeStruct((_HB, D), jnp.float32),
      mesh=mesh,
      scratch_types=[
          pltpu.VMEM((_CHUNK * S,), jnp.int32),
          pltpu.VMEM((S, _CHUNK), jnp.int32),
          pltpu.VMEM((_CHUNK, D), jnp.float32),
          pltpu.SemaphoreType.DMA,
          pltpu.SemaphoreType.DMA,
          pltpu.SemaphoreType.DMA,
          pltpu.SemaphoreType.DMA,
      ],
      compiler_params=pltpu.CompilerParams(needs_layout_passes=False),
  )


def _tc_matmul_kernel(x_ref, n_ref, ws_ref, wn_ref, o_ref):
  acc = jnp.dot(x_ref[...], ws_ref[...], preferred_element_type=jnp.float32)
  acc += jnp.dot(n_ref[...], wn_ref[...], preferred_element_type=jnp.float32)
  o_ref[...] = jnp.maximum(acc, 0.0)


_BM = 1024


def _make_mm(half):
  blk0 = half * (_HB // _BM)
  return pl.pallas_call(
      _tc_matmul_kernel,
      grid=(_HB // _BM,),
      in_specs=[
          pl.BlockSpec((_BM, D), lambda i: (i + blk0, 0)),
          pl.BlockSpec((_BM, D), lambda i: (i, 0)),
          pl.BlockSpec((D, EMB), lambda i: (0, 0)),
          pl.BlockSpec((D, EMB), lambda i: (0, 0)),
      ],
      out_specs=pl.BlockSpec((_BM, EMB), lambda i: (i, 0)),
      out_shape=jax.ShapeDtypeStruct((_HB, EMB), jnp.float32),
  )


@jax.jit
def kernel(nodes, feat_data, neigh_idx, W):
  nodes = nodes.astype(jnp.int32)
  nodes2 = nodes.reshape(B // 128, 128)

  mesh = plsc.VectorSubcoreMesh(core_axis_name="c", subcore_axis_name="s")
  sc_self = pl.kernel(
      _sc_self_kernel,
      out_type=jax.ShapeDtypeStruct((B, D), jnp.float32),
      mesh=mesh,
      scratch_types=[
          pltpu.VMEM((2, _CHUNK), jnp.int32),
          pltpu.VMEM((2, _CHUNK, D), jnp.float32),
          pltpu.SemaphoreType.DMA,
          pltpu.SemaphoreType.DMA,
      ],
  )
  self_feats = sc_self(feat_data, nodes2)

  nbf = neigh_idx.astype(jnp.int32).at[nodes].get(
      mode="promise_in_bounds").reshape(B * S)
  nsum_a = _make_nsum_kernel(0)(feat_data, nbf)
  nsum_b = _make_nsum_kernel(1)(feat_data, nbf)

  ws = W[:, :D].T
  wn = W[:, D:].T * jnp.float32(1.0 / S)
  out_a = _make_mm(0)(self_feats, nsum_a, ws, wn)
  out_b = _make_mm(1)(self_feats, nsum_b, ws, wn)
  return jnp.concatenate([out_a, out_b], axis=0)


# speedup vs baseline: 1.0746x; 1.0746x over previous
import functools

import jax
import jax.numpy as jnp
from jax import lax
from jax.experimental import pallas as pl
from jax.experimental.pallas import tpu as pltpu
from jax.experimental.pallas import tpu_sc as plsc

N = 50000
D = 128
S = 10
EMB = 128
B = 8192

_INFO = plsc.get_sparse_core_info()
_NC = _INFO.num_cores
_NS = _INFO.num_subcores
_NW = _NC * _NS
_CHUNK = 128
_HB = B // 2


def _make_sc_kernel(half):
  def body(feat_hbm, nodes_hbm, nb_hbm, self_out, nsum_out,
           nodes_v, nb_v, nbt_v, self_v, nsum_v,
           sem_self, sem_nb, sem_s0, sem_acc, sem_out):
    wid = lax.axis_index("s") * _NC + lax.axis_index("c")
    pltpu.sync_copy(nodes_hbm.at[pl.ds(half * (_HB // _CHUNK) + wid, 1)],
                    nodes_v)
    idx = nodes_v.at[0]
    cp_self = pltpu.async_copy(feat_hbm.at[idx], self_v, sem_self)
    pltpu.async_copy(
        nb_hbm.at[pl.ds(half * _HB + wid * _CHUNK, _CHUNK)], nb_v,
        sem_nb).wait()
    for s in range(S):
      col = jnp.full((16,), s, jnp.int32)
      for g in range(_CHUNK // 16):
        rows = lax.iota(jnp.int32, 16) + jnp.int32(g * 16)
        nbt_v[s, pl.ds(g * 16, 16)] = plsc.load_gather(nb_v, [rows, col])
    pltpu.async_copy(feat_hbm.at[nbt_v.at[0]], nsum_v, sem_s0).wait()
    cps = [pltpu.async_copy(feat_hbm.at[nbt_v.at[s]], nsum_v, sem_acc,
                            add=True) for s in range(1, S)]
    cp_self.wait()
    cpo = pltpu.async_copy(self_v, self_out.at[pl.ds(wid * _CHUNK, _CHUNK)],
                           sem_out)
    for cp in cps:
      cp.wait()
    cpo.wait()
    pltpu.async_copy(nsum_v, nsum_out.at[pl.ds(wid * _CHUNK, _CHUNK)],
                     sem_out).wait()

  mesh = plsc.VectorSubcoreMesh(core_axis_name="c", subcore_axis_name="s")
  return pl.kernel(
      body,
      out_type=(jax.ShapeDtypeStruct((_HB, D), jnp.float32),
                jax.ShapeDtypeStruct((_HB, D), jnp.float32)),
      mesh=mesh,
      scratch_types=[
          pltpu.VMEM((1, _CHUNK), jnp.int32),
          pltpu.VMEM((_CHUNK, S), jnp.int32),
          pltpu.VMEM((S, _CHUNK), jnp.int32),
          pltpu.VMEM((_CHUNK, D), jnp.float32),
          pltpu.VMEM((_CHUNK, D), jnp.float32),
          pltpu.SemaphoreType.DMA,
          pltpu.SemaphoreType.DMA,
          pltpu.SemaphoreType.DMA,
          pltpu.SemaphoreType.DMA,
          pltpu.SemaphoreType.DMA,
      ],
      compiler_params=pltpu.CompilerParams(needs_layout_passes=False),
  )


def _mm_first(x_ref, n_ref, ws_ref, wn_ref, o_ref):
  acc = jnp.dot(x_ref[...], ws_ref[...], preferred_element_type=jnp.float32)
  acc += jnp.dot(n_ref[...], wn_ref[...], preferred_element_type=jnp.float32)
  o_ref[...] = jnp.maximum(acc, 0.0)


def _mm_second(x_ref, n_ref, ws_ref, wn_ref, prev_ref, o_ref):
  del prev_ref
  acc = jnp.dot(x_ref[...], ws_ref[...], preferred_element_type=jnp.float32)
  acc += jnp.dot(n_ref[...], wn_ref[...], preferred_element_type=jnp.float32)
  o_ref[...] = jnp.maximum(acc, 0.0)


_BM = 1024


@jax.jit
def kernel(nodes, feat_data, neigh_idx, W):
  nodes = nodes.astype(jnp.int32)
  nodes2 = nodes.reshape(B // 128, 128)
  neigh_idx = neigh_idx.astype(jnp.int32)

  nb = neigh_idx.at[nodes].get(mode="promise_in_bounds")
  self_a, nsum_a = _make_sc_kernel(0)(feat_data, nodes2, nb)
  self_b, nsum_b = _make_sc_kernel(1)(feat_data, nodes2, nb)

  ws = W[:, :D].T
  wn = W[:, D:].T * jnp.float32(1.0 / S)

  w_spec = pl.BlockSpec((D, EMB), lambda i: (0, 0))
  half_spec = pl.BlockSpec((_BM, D), lambda i: (i, 0))
  out_a = pl.pallas_call(
      _mm_first,
      grid=(_HB // _BM,),
      in_specs=[half_spec, half_spec, w_spec, w_spec],
      out_specs=pl.BlockSpec((_BM, EMB), lambda i: (i, 0)),
      out_shape=jax.ShapeDtypeStruct((B, EMB), jnp.float32),
  )(self_a, nsum_a, ws, wn)
  out = pl.pallas_call(
      _mm_second,
      grid=(_HB // _BM,),
      in_specs=[half_spec, half_spec, w_spec, w_spec,
                pl.BlockSpec(memory_space=pl.ANY)],
      out_specs=pl.BlockSpec((_BM, EMB),
                             lambda i: (i + _HB // _BM, 0)),
      out_shape=jax.ShapeDtypeStruct((B, EMB), jnp.float32),
      input_output_aliases={4: 0},
  )(self_b, nsum_b, ws, wn, out_a)
  return out
